# baseline (device time: 15301 ns/iter reference)
import jax
import jax.numpy as jnp
from jax import lax
from jax.experimental import pallas as pl
from jax.experimental.pallas import tpu as pltpu

N_DEV = 16
N_TOK = 256
ROWS_PER_DEV = N_TOK // N_DEV
N_EXP = 32
EXP_PER_DEV = N_EXP // N_DEV
D_OUT = 256


def kernel(x, router_W, route_idx, expert_W, shared_W):
    def body(x_ref, rw_ref, idx_ref, ew_ref, sw_ref, idx_smem, out_ref,
             send_buf, gather_buf, send_sem, recv_sem):
        my = lax.axis_index("i")

        scores = jnp.dot(x_ref[:, :], rw_ref[:, :],
                         preferred_element_type=jnp.float32)
        s_max = jnp.max(scores, axis=-1, keepdims=True)
        ex = jnp.exp(scores - s_max)
        probs = ex / jnp.sum(ex, axis=-1, keepdims=True)
        idx = idx_ref[:, :]
        col = lax.broadcasted_iota(jnp.int32, (N_TOK, N_EXP), 1)
        p_sel = jnp.sum(jnp.where(col == idx, probs, 0.0),
                        axis=-1, keepdims=True)

        e0 = my * EXP_PER_DEV
        m0 = jnp.where(idx == e0, p_sel, 0.0)
        m1 = jnp.where(idx == e0 + 1, p_sel, 0.0)
        xv = x_ref[:, :]
        send_buf[:, :] = (
            jnp.dot(m0 * xv, ew_ref[0, :, :],
                    preferred_element_type=jnp.float32)
            + jnp.dot(m1 * xv, ew_ref[1, :, :],
                      preferred_element_type=jnp.float32)
        )

        owner = idx // EXP_PER_DEV
        rowdev = lax.broadcasted_iota(jnp.int32, (N_TOK, 1), 0) // ROWS_PER_DEV
        inbound = jnp.logical_and(rowdev == my, owner != my)
        n_in = jnp.sum(inbound.astype(jnp.int32))
        outbound = jnp.logical_and(owner == my, rowdev != my)
        n_out = jnp.sum(outbound.astype(jnp.int32))

        barrier_sem = pltpu.get_barrier_semaphore()
        for o in range(1, N_DEV):
            peer = lax.rem(my + o, N_DEV)
            pl.semaphore_signal(barrier_sem, inc=1, device_id=(peer,),
                                device_id_type=pl.DeviceIdType.MESH)
        pl.semaphore_wait(barrier_sem, N_DEV - 1)

        for i in range(N_TOK):
            dest = i // ROWS_PER_DEV
            slot = i % ROWS_PER_DEV
            row_owner = idx_smem[i, 0] // EXP_PER_DEV

            @pl.when(jnp.logical_and(row_owner == my, dest != my))
            def _(i=i, dest=dest, slot=slot):
                pltpu.make_async_remote_copy(
                    src_ref=send_buf.at[pl.ds(i, 1), :],
                    dst_ref=gather_buf.at[pl.ds(slot, 1), :],
                    send_sem=send_sem,
                    recv_sem=recv_sem,
                    device_id=(dest,),
                    device_id_type=pl.DeviceIdType.MESH,
                ).start()

        x_my = x_ref[pl.ds(my * ROWS_PER_DEV, ROWS_PER_DEV), :]
        shared = jnp.dot(x_my, sw_ref[:, :], preferred_element_type=jnp.float32)

        dummy = pltpu.make_async_remote_copy(
            src_ref=send_buf.at[pl.ds(0, 1), :],
            dst_ref=gather_buf.at[pl.ds(0, 1), :],
            send_sem=send_sem,
            recv_sem=recv_sem,
            device_id=(0,),
            device_id_type=pl.DeviceIdType.MESH,
        )
        lax.fori_loop(0, n_in, lambda t, c: (dummy.wait_recv(), c)[1], 0)
        my_mask = (
            idx_ref[pl.ds(my * ROWS_PER_DEV, ROWS_PER_DEV), :] // EXP_PER_DEV
            == my
        )
        own_rows = send_buf[pl.ds(my * ROWS_PER_DEV, ROWS_PER_DEV), :]
        out_ref[:, :] = shared + jnp.where(my_mask, own_rows,
                                           gather_buf[:, :])

        lax.fori_loop(0, n_out, lambda t, c: (dummy.wait_send(), c)[1], 0)

    return pl.pallas_call(
        body,
        out_shape=jax.ShapeDtypeStruct((ROWS_PER_DEV, D_OUT), jnp.float32),
        in_specs=[pl.BlockSpec(memory_space=pltpu.VMEM)] * 5
        + [pl.BlockSpec(memory_space=pltpu.SMEM)],
        out_specs=pl.BlockSpec(memory_space=pltpu.VMEM),
        scratch_shapes=[
            pltpu.VMEM((N_TOK, D_OUT), jnp.float32),
            pltpu.VMEM((ROWS_PER_DEV, D_OUT), jnp.float32),
            pltpu.SemaphoreType.DMA,
            pltpu.SemaphoreType.DMA,
        ],
        compiler_params=pltpu.CompilerParams(collective_id=0),
    )(x, router_W, route_idx, expert_W, shared_W, route_idx)


# device time: 13268 ns/iter; 1.1532x vs baseline; 1.1532x over previous
import jax
import jax.numpy as jnp
from jax import lax
from jax.experimental import pallas as pl
from jax.experimental.pallas import tpu as pltpu

N_DEV = 16
N_TOK = 256
ROWS_PER_DEV = N_TOK // N_DEV
N_EXP = 32
EXP_PER_DEV = N_EXP // N_DEV
D_OUT = 256
HALF = N_TOK // 2


def kernel(x, router_W, route_idx, expert_W, shared_W):
    def body(x_ref, rw_ref, idx_ref, ew_ref, sw_ref, out_ref,
             send_buf, recv_buf, send_sems, recv_sems):
        my = lax.axis_index("i")

        barrier_sem = pltpu.get_barrier_semaphore()
        for o in range(1, N_DEV):
            peer = lax.rem(my + o, N_DEV)
            pl.semaphore_signal(barrier_sem, inc=1, device_id=(peer,),
                                device_id_type=pl.DeviceIdType.MESH)
        pl.semaphore_wait(barrier_sem, N_DEV - 1)

        scores = jnp.dot(x_ref[:, :], rw_ref[:, :],
                         preferred_element_type=jnp.float32)
        s_max = jnp.max(scores, axis=-1, keepdims=True)
        ex = jnp.exp(scores - s_max)
        probs = ex / jnp.sum(ex, axis=-1, keepdims=True)
        idx = idx_ref[:, :]
        col = lax.broadcasted_iota(jnp.int32, (N_TOK, N_EXP), 1)
        p_sel = jnp.sum(jnp.where(col == idx, probs, 0.0),
                        axis=-1, keepdims=True)

        e0 = my * EXP_PER_DEV
        m0 = jnp.where(idx == e0, p_sel, 0.0)
        m1 = jnp.where(idx == e0 + 1, p_sel, 0.0)
        xv = x_ref[:, :]
        w0 = ew_ref[0, :, :].astype(jnp.bfloat16)
        w1 = ew_ref[1, :, :].astype(jnp.bfloat16)

        def contrib_rows(lo):
            xs = xv[lo:lo + HALF, :]
            a = (m0[lo:lo + HALF, :] * xs).astype(jnp.bfloat16)
            b = (m1[lo:lo + HALF, :] * xs).astype(jnp.bfloat16)
            acc = (jnp.dot(a, w0, preferred_element_type=jnp.float32)
                   + jnp.dot(b, w1, preferred_element_type=jnp.float32))
            return acc.astype(jnp.bfloat16)

        def start_sends(half, rdmas):
            for o in range(1, N_DEV):
                dest = lax.rem(my + o, N_DEV)
                rdma = pltpu.make_async_remote_copy(
                    src_ref=send_buf.at[pl.ds(dest * ROWS_PER_DEV,
                                              ROWS_PER_DEV), :],
                    dst_ref=recv_buf.at[o],
                    send_sem=send_sems.at[o],
                    recv_sem=recv_sems.at[o],
                    device_id=(dest,),
                    device_id_type=pl.DeviceIdType.MESH,
                )
                in_half = (dest < N_DEV // 2) if half == 0 else (
                    dest >= N_DEV // 2)
                pl.when(in_half)(rdma.start)
                rdmas.append(rdma)

        rdmas_a, rdmas_b = [], []
        send_buf[pl.ds(0, HALF), :] = contrib_rows(0)
        start_sends(0, rdmas_a)
        send_buf[pl.ds(HALF, HALF), :] = contrib_rows(HALF)
        start_sends(1, rdmas_b)

        x_my = x_ref[pl.ds(my * ROWS_PER_DEV, ROWS_PER_DEV), :]
        shared = jnp.dot(x_my.astype(jnp.bfloat16),
                         sw_ref[:, :].astype(jnp.bfloat16),
                         preferred_element_type=jnp.float32)

        for ra in rdmas_a:
            ra.wait_recv()

        own = send_buf[pl.ds(my * ROWS_PER_DEV, ROWS_PER_DEV), :]
        out_ref[:, :] = (
            shared + own.astype(jnp.float32)
            + jnp.sum(recv_buf[1:, :, :].astype(jnp.float32), axis=0)
        )

        for ra in rdmas_a:
            ra.wait_send()

    return pl.pallas_call(
        body,
        out_shape=jax.ShapeDtypeStruct((ROWS_PER_DEV, D_OUT), jnp.float32),
        in_specs=[pl.BlockSpec(memory_space=pltpu.VMEM)] * 5,
        out_specs=pl.BlockSpec(memory_space=pltpu.VMEM),
        scratch_shapes=[
            pltpu.VMEM((N_TOK, D_OUT), jnp.bfloat16),
            pltpu.VMEM((N_DEV, ROWS_PER_DEV, D_OUT), jnp.bfloat16),
            pltpu.SemaphoreType.DMA((N_DEV,)),
            pltpu.SemaphoreType.DMA((N_DEV,)),
        ],
        compiler_params=pltpu.CompilerParams(collective_id=0),
    )(x, router_W, route_idx, expert_W, shared_W)


# device time: 12895 ns/iter; 1.1866x vs baseline; 1.0289x over previous
import jax
import jax.numpy as jnp
from jax import lax
from jax.experimental import pallas as pl
from jax.experimental.pallas import tpu as pltpu

N_DEV = 16
N_TOK = 256
ROWS_PER_DEV = N_TOK // N_DEV
N_EXP = 32
EXP_PER_DEV = N_EXP // N_DEV
D_OUT = 256
HALF = N_TOK // 2


def kernel(x, router_W, route_idx, expert_W, shared_W):
    def body(x_ref, rw_ref, idx_ref, ew_any, sw_any, out_any,
             send_buf, recv_buf, ew_vmem, sw_vmem, out_vmem,
             send_sems, recv_sems, load_sems, store_sem):
        my = lax.axis_index("i")

        barrier_sem = pltpu.get_barrier_semaphore()
        for o in range(1, N_DEV):
            peer = lax.rem(my + o, N_DEV)
            pl.semaphore_signal(barrier_sem, inc=1, device_id=(peer,),
                                device_id_type=pl.DeviceIdType.MESH)

        ew_copy = pltpu.make_async_copy(ew_any, ew_vmem, load_sems.at[0])
        sw_copy = pltpu.make_async_copy(sw_any, sw_vmem, load_sems.at[1])
        ew_copy.start()
        sw_copy.start()

        scores = jnp.dot(x_ref[:, :], rw_ref[:, :],
                         preferred_element_type=jnp.float32)
        s_max = jnp.max(scores, axis=-1, keepdims=True)
        ex = jnp.exp(scores - s_max)
        probs = ex / jnp.sum(ex, axis=-1, keepdims=True)
        idx = idx_ref[:, :]
        col = lax.broadcasted_iota(jnp.int32, (N_TOK, N_EXP), 1)
        p_sel = jnp.sum(jnp.where(col == idx, probs, 0.0),
                        axis=-1, keepdims=True)

        e0 = my * EXP_PER_DEV
        m0 = jnp.where(idx == e0, p_sel, 0.0)
        m1 = jnp.where(idx == e0 + 1, p_sel, 0.0)
        xv = x_ref[:, :]

        pl.semaphore_wait(barrier_sem, N_DEV - 1)
        ew_copy.wait()
        w0 = ew_vmem[0, :, :].astype(jnp.bfloat16)
        w1 = ew_vmem[1, :, :].astype(jnp.bfloat16)

        def contrib_rows(lo):
            xs = xv[lo:lo + HALF, :]
            a = (m0[lo:lo + HALF, :] * xs).astype(jnp.bfloat16)
            b = (m1[lo:lo + HALF, :] * xs).astype(jnp.bfloat16)
            acc = (jnp.dot(a, w0, preferred_element_type=jnp.float32)
                   + jnp.dot(b, w1, preferred_element_type=jnp.float32))
            return acc.astype(jnp.bfloat16)

        def start_sends(half, rdmas):
            for o in range(1, N_DEV):
                dest = lax.rem(my + o, N_DEV)
                rdma = pltpu.make_async_remote_copy(
                    src_ref=send_buf.at[pl.ds(dest * ROWS_PER_DEV,
                                              ROWS_PER_DEV), :],
                    dst_ref=recv_buf.at[o],
                    send_sem=send_sems.at[o],
                    recv_sem=recv_sems.at[o],
                    device_id=(dest,),
                    device_id_type=pl.DeviceIdType.MESH,
                )
                in_half = (dest < N_DEV // 2) if half == 0 else (
                    dest >= N_DEV // 2)
                pl.when(in_half)(rdma.start)
                rdmas.append(rdma)

        rdmas_a, rdmas_b = [], []
        send_buf[pl.ds(0, HALF), :] = contrib_rows(0)
        start_sends(0, rdmas_a)
        send_buf[pl.ds(HALF, HALF), :] = contrib_rows(HALF)
        start_sends(1, rdmas_b)

        sw_copy.wait()
        x_my = x_ref[pl.ds(my * ROWS_PER_DEV, ROWS_PER_DEV), :]
        shared = jnp.dot(x_my.astype(jnp.bfloat16),
                         sw_vmem[:, :].astype(jnp.bfloat16),
                         preferred_element_type=jnp.float32)

        for ra in rdmas_a:
            ra.wait_recv()

        own = send_buf[pl.ds(my * ROWS_PER_DEV, ROWS_PER_DEV), :]
        out_vmem[:, :] = (
            shared + own.astype(jnp.float32)
            + jnp.sum(recv_buf[1:, :, :].astype(jnp.float32), axis=0)
        )
        out_copy = pltpu.make_async_copy(out_vmem, out_any, store_sem)
        out_copy.start()

        for ra in rdmas_a:
            ra.wait_send()
        out_copy.wait()

    return pl.pallas_call(
        body,
        out_shape=jax.ShapeDtypeStruct((ROWS_PER_DEV, D_OUT), jnp.float32),
        in_specs=[
            pl.BlockSpec(memory_space=pltpu.VMEM),
            pl.BlockSpec(memory_space=pltpu.VMEM),
            pl.BlockSpec(memory_space=pltpu.VMEM),
            pl.BlockSpec(memory_space=pltpu.MemorySpace.HBM),
            pl.BlockSpec(memory_space=pltpu.MemorySpace.HBM),
        ],
        out_specs=pl.BlockSpec(memory_space=pltpu.MemorySpace.HBM),
        scratch_shapes=[
            pltpu.VMEM((N_TOK, D_OUT), jnp.bfloat16),
            pltpu.VMEM((N_DEV, ROWS_PER_DEV, D_OUT), jnp.bfloat16),
            pltpu.VMEM((EXP_PER_DEV, 128, D_OUT), jnp.float32),
            pltpu.VMEM((128, D_OUT), jnp.float32),
            pltpu.VMEM((ROWS_PER_DEV, D_OUT), jnp.float32),
            pltpu.SemaphoreType.DMA((N_DEV,)),
            pltpu.SemaphoreType.DMA((N_DEV,)),
            pltpu.SemaphoreType.DMA((2,)),
            pltpu.SemaphoreType.DMA,
        ],
        compiler_params=pltpu.CompilerParams(collective_id=0),
    )(x, router_W, route_idx, expert_W, shared_W)


# device time: 12503 ns/iter; 1.2238x vs baseline; 1.0314x over previous
import jax
import jax.numpy as jnp
from jax import lax
from jax.experimental import pallas as pl
from jax.experimental.pallas import tpu as pltpu

N_DEV = 16
N_TOK = 256
ROWS_PER_DEV = N_TOK // N_DEV
N_EXP = 32
EXP_PER_DEV = N_EXP // N_DEV
D_OUT = 256
HALF = N_TOK // 2


def kernel(x, router_W, route_idx, expert_W, shared_W):
    def body(x_ref, rw_ref, idx_ref, ew_vmem, sw_vmem, out_ref,
             send_buf, recv_buf, send_sems, recv_sems):
        my = lax.axis_index("i")

        barrier_sem = pltpu.get_barrier_semaphore()
        for o in range(1, N_DEV):
            peer = lax.rem(my + o, N_DEV)
            pl.semaphore_signal(barrier_sem, inc=1, device_id=(peer,),
                                device_id_type=pl.DeviceIdType.MESH)

        scores = jnp.dot(x_ref[:, :], rw_ref[:, :],
                         preferred_element_type=jnp.float32)
        s_max = jnp.max(scores, axis=-1, keepdims=True)
        ex = jnp.exp(scores - s_max)
        probs = ex / jnp.sum(ex, axis=-1, keepdims=True)
        idx = idx_ref[:, :]
        col = lax.broadcasted_iota(jnp.int32, (N_TOK, N_EXP), 1)
        p_sel = jnp.sum(jnp.where(col == idx, probs, 0.0),
                        axis=-1, keepdims=True)

        e0 = my * EXP_PER_DEV
        m0 = jnp.where(idx == e0, p_sel, 0.0)
        m1 = jnp.where(idx == e0 + 1, p_sel, 0.0)
        xv = x_ref[:, :]

        w0 = ew_vmem[0, :, :].astype(jnp.bfloat16)
        w1 = ew_vmem[1, :, :].astype(jnp.bfloat16)

        def contrib_rows(lo):
            xs = xv[lo:lo + HALF, :]
            a = (m0[lo:lo + HALF, :] * xs).astype(jnp.bfloat16)
            b = (m1[lo:lo + HALF, :] * xs).astype(jnp.bfloat16)
            acc = (jnp.dot(a, w0, preferred_element_type=jnp.float32)
                   + jnp.dot(b, w1, preferred_element_type=jnp.float32))
            return acc.astype(jnp.bfloat16)

        def start_sends(half, rdmas):
            for o in range(1, N_DEV):
                dest = lax.rem(my + o, N_DEV)
                rdma = pltpu.make_async_remote_copy(
                    src_ref=send_buf.at[pl.ds(dest * ROWS_PER_DEV,
                                              ROWS_PER_DEV), :],
                    dst_ref=recv_buf.at[o],
                    send_sem=send_sems.at[o],
                    recv_sem=recv_sems.at[o],
                    device_id=(dest,),
                    device_id_type=pl.DeviceIdType.MESH,
                )
                in_half = (dest < N_DEV // 2) if half == 0 else (
                    dest >= N_DEV // 2)
                pl.when(in_half)(rdma.start)
                rdmas.append(rdma)

        rdmas_a, rdmas_b = [], []
        send_buf[pl.ds(0, HALF), :] = contrib_rows(0)
        pl.semaphore_wait(barrier_sem, N_DEV - 1)
        start_sends(0, rdmas_a)
        send_buf[pl.ds(HALF, HALF), :] = contrib_rows(HALF)
        start_sends(1, rdmas_b)

        x_my = x_ref[pl.ds(my * ROWS_PER_DEV, ROWS_PER_DEV), :]
        shared = jnp.dot(x_my.astype(jnp.bfloat16),
                         sw_vmem[:, :].astype(jnp.bfloat16),
                         preferred_element_type=jnp.float32)

        for ra in rdmas_a:
            ra.wait_recv()

        own = send_buf[pl.ds(my * ROWS_PER_DEV, ROWS_PER_DEV), :]
        out_ref[:, :] = (
            shared + own.astype(jnp.float32)
            + jnp.sum(recv_buf[1:, :, :].astype(jnp.float32), axis=0)
        )

        for ra in rdmas_a:
            ra.wait_send()

    return pl.pallas_call(
        body,
        out_shape=jax.ShapeDtypeStruct((ROWS_PER_DEV, D_OUT), jnp.float32),
        in_specs=[
            pl.BlockSpec(memory_space=pltpu.VMEM),
            pl.BlockSpec(memory_space=pltpu.VMEM),
            pl.BlockSpec(memory_space=pltpu.VMEM),
            pl.BlockSpec(memory_space=pltpu.VMEM),
            pl.BlockSpec(memory_space=pltpu.VMEM),
        ],
        out_specs=pl.BlockSpec(memory_space=pltpu.VMEM),
        scratch_shapes=[
            pltpu.VMEM((N_TOK, D_OUT), jnp.bfloat16),
            pltpu.VMEM((N_DEV, ROWS_PER_DEV, D_OUT), jnp.bfloat16),
            pltpu.SemaphoreType.DMA((N_DEV,)),
            pltpu.SemaphoreType.DMA((N_DEV,)),
        ],
        compiler_params=pltpu.CompilerParams(collective_id=0),
    )(x, router_W, route_idx, expert_W, shared_W)
